# baseline (speedup 1.0000x reference)
import functools

import jax
import jax.numpy as jnp
from jax import lax
from jax.experimental import pallas as pl
from jax.experimental.pallas import tpu as pltpu
from jax.experimental.pallas import tpu_sc as plsc

N = 10000
NP = N // 2
E = 320000
D_HID = 64
NG = 16
NC, NS = 2, 16
NW = NC * NS
CH = 128
NCHUNK = E // CH
CPT = NCHUNK // NW
XTRA = NCHUNK - NW * CPT
RB = 8
KB = 5
RPA = 624
NBINS = 10240


def _sc_mesh():
    return plsc.VectorSubcoreMesh(core_axis_name="c", subcore_axis_name="s")


def _deg_hist(ev):
    @functools.partial(
        pl.kernel,
        out_type=jax.ShapeDtypeStruct((NW, NBINS), jnp.float32),
        mesh=_sc_mesh(),
        scratch_types=[
            pltpu.VMEM((CPT, 2, CH), jnp.int32),
            pltpu.VMEM((1, 2, CH), jnp.int32),
            pltpu.VMEM((NBINS,), jnp.float32),
        ],
        compiler_params=pltpu.CompilerParams(
            needs_layout_passes=False, use_tc_tiling_on_sc=False
        ),
    )
    def k(ev_hbm, out_hbm, ebuf, xbuf, hist):
        c = lax.axis_index("c")
        s = lax.axis_index("s")
        wid = c * NS + s
        pltpu.sync_copy(ev_hbm.at[pl.ds(wid * CPT, CPT)], ebuf)

        @pl.when(wid < XTRA)
        def _():
            pltpu.sync_copy(ev_hbm.at[pl.ds(NW * CPT + wid, 1)], xbuf)

        zeros = jnp.zeros((16,), jnp.float32)

        @pl.loop(0, NBINS // 16)
        def _(r):
            hist[pl.ds(r * 16, 16)] = zeros

        ones = jnp.ones((16,), jnp.float32)

        @pl.loop(0, CPT)
        def _(g):
            for j in range(CH // 16):
                idx = ebuf[g, 1, pl.ds(j * 16, 16)]
                plsc.addupdate_scatter(hist, [idx], ones)

        @pl.when(wid < XTRA)
        def _():
            for j in range(CH // 16):
                idx = xbuf[0, 1, pl.ds(j * 16, 16)]
                plsc.addupdate_scatter(hist, [idx], ones)

        pltpu.sync_copy(hist, out_hbm.at[wid])

    return k(ev)


def _edge_agg(h, ev):
    @functools.partial(
        pl.kernel,
        out_type=jax.ShapeDtypeStruct((NC, N, D_HID), jnp.float32),
        mesh=_sc_mesh(),
        scratch_types=[
            pltpu.VMEM((CPT, 2, CH), jnp.int32),
            pltpu.VMEM((1, 2, CH), jnp.int32),
            pltpu.VMEM((RB, CH, D_HID), jnp.float32),
            pltpu.VMEM((48, D_HID), jnp.float32),
            pltpu.VMEM_SHARED((N, D_HID), jnp.float32),
            pltpu.SemaphoreType.DMA,
            pltpu.SemaphoreType.DMA,
        ],
        compiler_params=pltpu.CompilerParams(
            needs_layout_passes=False, use_tc_tiling_on_sc=False
        ),
    )
    def k(h_hbm, ev_hbm, out_hbm, ebuf, xbuf, rows, zbuf, acc, sem, ssem):
        c = lax.axis_index("c")
        s = lax.axis_index("s")
        wid = c * NS + s
        pltpu.sync_copy(ev_hbm.at[pl.ds(wid * CPT, CPT)], ebuf)

        @pl.when(wid < XTRA)
        def _():
            pltpu.sync_copy(ev_hbm.at[pl.ds(NW * CPT + wid, 1)], xbuf)

        zeros = jnp.zeros((16,), jnp.float32)

        @pl.loop(0, 48)
        def _(r):
            for j in range(D_HID // 16):
                zbuf[r, pl.ds(j * 16, 16)] = zeros

        for j in range(13):
            pltpu.async_copy(zbuf, acc.at[pl.ds(s * RPA + j * 48, 48)], sem)

        @pl.when(s == NS - 1)
        def _():
            pltpu.async_copy(zbuf.at[pl.ds(0, N - NS * RPA)],
                            acc.at[pl.ds(NS * RPA, N - NS * RPA)], sem)

        for j in range(13):
            pltpu.make_async_copy(
                zbuf, acc.at[pl.ds(s * RPA + j * 48, 48)], sem
            ).wait()

        @pl.when(s == NS - 1)
        def _():
            pltpu.make_async_copy(
                zbuf.at[pl.ds(0, N - NS * RPA)],
                acc.at[pl.ds(NS * RPA, N - NS * RPA)], sem
            ).wait()

        plsc.subcore_barrier()

        for b in range(KB):
            pltpu.async_copy(h_hbm.at[ebuf.at[b, 0]], rows.at[b], sem)

        @pl.loop(0, CPT, step=RB)
        def _(g):
            for b in range(RB):
                cur = g + b

                @pl.when(cur < CPT)
                def _():
                    pltpu.make_async_copy(
                        h_hbm.at[ebuf.at[cur, 0]], rows.at[b], sem
                    ).wait()
                    pltpu.async_copy(
                        rows.at[b], acc.at[ebuf.at[cur, 1]], ssem, add=True
                    )

                    @pl.when(cur >= RB - KB)
                    def _():
                        old = cur + KB - RB
                        pltpu.make_async_copy(
                            rows.at[(KB + b) % RB], acc.at[ebuf.at[old, 1]],
                            ssem,
                        ).wait()

                    @pl.when(cur + KB < CPT)
                    def _():
                        pltpu.async_copy(
                            h_hbm.at[ebuf.at[cur + KB, 0]],
                            rows.at[(KB + b) % RB], sem,
                        )

        for j in range(RB - KB):
            old = CPT - (RB - KB) + j
            pltpu.make_async_copy(
                rows.at[old % RB], acc.at[ebuf.at[old, 1]], ssem
            ).wait()

        @pl.when(wid < XTRA)
        def _():
            pltpu.async_copy(h_hbm.at[xbuf.at[0, 0]], rows.at[0], sem).wait()
            pltpu.sync_copy(rows.at[0], acc.at[xbuf.at[0, 1]], add=True)

        plsc.subcore_barrier()
        pltpu.sync_copy(acc.at[pl.ds(s * RPA, RPA)], out_hbm.at[c, pl.ds(s * RPA, RPA)])

        @pl.when(s == NS - 1)
        def _():
            pltpu.sync_copy(acc.at[pl.ds(NS * RPA, N - NS * RPA)],
                            out_hbm.at[c, pl.ds(NS * RPA, N - NS * RPA)])

    return k(h, ev)


def _tc_dis(histp):
    def body(hp_ref, dis_ref):
        hp = hp_ref[...].reshape(NW, NBINS // 128, 128)
        deg = jnp.sum(hp, axis=0) + 1.0
        dis_ref[...] = lax.rsqrt(deg)

    return pl.pallas_call(
        body,
        out_shape=jax.ShapeDtypeStruct((NBINS // 128, 128), jnp.float32),
    )(histp)


def _dis_packed(dis2_ref):
    d = dis2_ref[...]
    lo = jnp.broadcast_to(d[:, 0:1], (NP, D_HID))
    hi = jnp.broadcast_to(d[:, 1:2], (NP, D_HID))
    return jnp.concatenate([lo, hi], axis=1)


def _tc_layer1(dis2, xp, W1p):
    def body(dis2_ref, xp_ref, w_ref, out_ref):
        out_ref[...] = _dis_packed(dis2_ref) * jnp.dot(
            xp_ref[...], w_ref[...], preferred_element_type=jnp.float32
        )

    return pl.pallas_call(
        body,
        out_shape=jax.ShapeDtypeStruct((NP, 128), jnp.float32),
    )(dis2, xp, W1p)


def _tc_layer2(aggp, h1p, dis2, b1p, W2p):
    def body(aggp_ref, h1p_ref, dis2_ref, b1p_ref, w_ref, out_ref):
        dis = _dis_packed(dis2_ref)
        agg = aggp_ref[0] + aggp_ref[1]
        h1 = jnp.maximum(dis * (agg + h1p_ref[...]) + b1p_ref[...], 0.0)
        out_ref[...] = dis * jnp.dot(
            h1, w_ref[...], preferred_element_type=jnp.float32
        )

    return pl.pallas_call(
        body,
        out_shape=jax.ShapeDtypeStruct((NP, 128), jnp.float32),
    )(aggp, h1p, dis2, b1p, W2p)


def _tc_final(aggp, h2p, dis2, b2p, batch2, linW, linb):
    def body(aggp_ref, h2p_ref, dis2_ref, b2p_ref, batch_ref, lw_ref, lb_ref,
             out_ref):
        dis = _dis_packed(dis2_ref)
        agg = aggp_ref[0] + aggp_ref[1]
        h2 = jnp.maximum(dis * (agg + h2p_ref[...]) + b2p_ref[...], 0.0)
        gid = lax.broadcasted_iota(jnp.int32, (NG, NP), 0)
        one_e = (batch_ref[0:1, :] == gid).astype(jnp.float32)
        one_o = (batch_ref[1:2, :] == gid).astype(jnp.float32)
        pooled = (
            jnp.dot(one_e, h2[:, :D_HID], preferred_element_type=jnp.float32)
            + jnp.dot(one_o, h2[:, D_HID:], preferred_element_type=jnp.float32)
        )
        counts = jnp.sum(one_e, axis=1, keepdims=True) + jnp.sum(
            one_o, axis=1, keepdims=True
        )
        mean = pooled / jnp.maximum(counts, 1.0)
        logits = jnp.dot(mean, lw_ref[...], preferred_element_type=jnp.float32)
        logits = logits + lb_ref[...]
        m = jnp.max(logits, axis=1, keepdims=True)
        lse = m + jnp.log(jnp.sum(jnp.exp(logits - m), axis=1, keepdims=True))
        out_ref[...] = logits - lse

    return pl.pallas_call(
        body,
        out_shape=jax.ShapeDtypeStruct((NG, 2), jnp.float32),
    )(aggp, h2p, dis2, b2p, batch2, linW, linb)


def _block_diag2(W):
    a, b = W.shape
    z = jnp.zeros((a, b), W.dtype)
    return jnp.concatenate(
        [jnp.concatenate([W, z], axis=1), jnp.concatenate([z, W], axis=1)],
        axis=0,
    )


def kernel(x, edge_index, batch, W1, b1, W2, b2, linW, linb):
    ev = jnp.transpose(
        edge_index.astype(jnp.int32).reshape(2, NCHUNK, CH), (1, 0, 2)
    )

    histp = _deg_hist(ev)
    dis80 = _tc_dis(histp.reshape(NW * NBINS // 128, 128))
    dis2 = dis80.reshape(NBINS)[:N].reshape(NP, 2)

    xp = x.reshape(NP, 2 * 128)
    W1p = _block_diag2(W1)
    W2p = _block_diag2(W2)
    b1p = jnp.concatenate([b1, b1]).reshape(1, 128)
    b2p = jnp.concatenate([b2, b2]).reshape(1, 128)
    batch2 = batch.astype(jnp.int32).reshape(NP, 2).T

    h1p = _tc_layer1(dis2, xp, W1p)
    agg1 = _edge_agg(h1p.reshape(N, D_HID), ev)
    agg1 = agg1.reshape(NC, NP, 128)
    h2p = _tc_layer2(agg1, h1p, dis2, b1p, W2p)
    agg2 = _edge_agg(h2p.reshape(N, D_HID), ev)
    agg2 = agg2.reshape(NC, NP, 128)
    out = _tc_final(agg2, h2p, dis2, b2p, batch2, linW, linb)
    return out


# speedup vs baseline: 1.0019x; 1.0019x over previous
import functools

import jax
import jax.numpy as jnp
from jax import lax
from jax.experimental import pallas as pl
from jax.experimental.pallas import tpu as pltpu
from jax.experimental.pallas import tpu_sc as plsc

N = 10000
NP = N // 2
E = 320000
D_HID = 64
NG = 16
NC, NS = 2, 16
NW = NC * NS
CH = 128
NCHUNK = E // CH
CPT = NCHUNK // NW
XTRA = NCHUNK - NW * CPT
RB = 8
RPA = 624
NBINS = 10240


def _sc_mesh():
    return plsc.VectorSubcoreMesh(core_axis_name="c", subcore_axis_name="s")


def _deg_hist(ev):
    @functools.partial(
        pl.kernel,
        out_type=jax.ShapeDtypeStruct((NW, NBINS), jnp.float32),
        mesh=_sc_mesh(),
        scratch_types=[
            pltpu.VMEM((CPT, 2, CH), jnp.int32),
            pltpu.VMEM((1, 2, CH), jnp.int32),
            pltpu.VMEM((NBINS,), jnp.float32),
        ],
        compiler_params=pltpu.CompilerParams(
            needs_layout_passes=False, use_tc_tiling_on_sc=False
        ),
    )
    def k(ev_hbm, out_hbm, ebuf, xbuf, hist):
        c = lax.axis_index("c")
        s = lax.axis_index("s")
        wid = c * NS + s
        pltpu.sync_copy(ev_hbm.at[pl.ds(wid * CPT, CPT)], ebuf)

        @pl.when(wid < XTRA)
        def _():
            pltpu.sync_copy(ev_hbm.at[pl.ds(NW * CPT + wid, 1)], xbuf)

        zeros = jnp.zeros((16,), jnp.float32)

        @pl.loop(0, NBINS // 16)
        def _(r):
            hist[pl.ds(r * 16, 16)] = zeros

        ones = jnp.ones((16,), jnp.float32)

        @pl.loop(0, CPT)
        def _(g):
            for j in range(CH // 16):
                idx = ebuf[g, 1, pl.ds(j * 16, 16)]
                plsc.addupdate_scatter(hist, [idx], ones)

        @pl.when(wid < XTRA)
        def _():
            for j in range(CH // 16):
                idx = xbuf[0, 1, pl.ds(j * 16, 16)]
                plsc.addupdate_scatter(hist, [idx], ones)

        pltpu.sync_copy(hist, out_hbm.at[wid])

    return k(ev)


def _edge_agg(h, ev):
    @functools.partial(
        pl.kernel,
        out_type=jax.ShapeDtypeStruct((NC, N, D_HID), jnp.float32),
        mesh=_sc_mesh(),
        scratch_types=[
            pltpu.VMEM((CPT, 2, CH), jnp.int32),
            pltpu.VMEM((1, 2, CH), jnp.int32),
            pltpu.VMEM((RB, CH, D_HID), jnp.float32),
            pltpu.VMEM((48, D_HID), jnp.float32),
            pltpu.VMEM_SHARED((N, D_HID), jnp.float32),
            pltpu.SemaphoreType.DMA,
        ],
        compiler_params=pltpu.CompilerParams(
            needs_layout_passes=False, use_tc_tiling_on_sc=False
        ),
    )
    def k(h_hbm, ev_hbm, out_hbm, ebuf, xbuf, rows, zbuf, acc, sem):
        c = lax.axis_index("c")
        s = lax.axis_index("s")
        wid = c * NS + s
        pltpu.sync_copy(ev_hbm.at[pl.ds(wid * CPT, CPT)], ebuf)

        @pl.when(wid < XTRA)
        def _():
            pltpu.sync_copy(ev_hbm.at[pl.ds(NW * CPT + wid, 1)], xbuf)

        zeros = jnp.zeros((16,), jnp.float32)

        @pl.loop(0, 48)
        def _(r):
            for j in range(D_HID // 16):
                zbuf[r, pl.ds(j * 16, 16)] = zeros

        for j in range(13):
            pltpu.async_copy(zbuf, acc.at[pl.ds(s * RPA + j * 48, 48)], sem)

        @pl.when(s == NS - 1)
        def _():
            pltpu.async_copy(zbuf.at[pl.ds(0, N - NS * RPA)],
                            acc.at[pl.ds(NS * RPA, N - NS * RPA)], sem)

        for j in range(13):
            pltpu.make_async_copy(
                zbuf, acc.at[pl.ds(s * RPA + j * 48, 48)], sem
            ).wait()

        @pl.when(s == NS - 1)
        def _():
            pltpu.make_async_copy(
                zbuf.at[pl.ds(0, N - NS * RPA)],
                acc.at[pl.ds(NS * RPA, N - NS * RPA)], sem
            ).wait()

        plsc.subcore_barrier()

        for b in range(RB):
            pltpu.async_copy(h_hbm.at[ebuf.at[b, 0]], rows.at[b], sem)

        @pl.loop(0, CPT, step=RB)
        def _(g):
            for b in range(RB):
                cur = g + b

                @pl.when(cur < CPT)
                def _():
                    pltpu.make_async_copy(
                        h_hbm.at[ebuf.at[cur, 0]], rows.at[b], sem
                    ).wait()
                    pltpu.sync_copy(rows.at[b], acc.at[ebuf.at[cur, 1]], add=True)

                    @pl.when(cur + RB < CPT)
                    def _():
                        pltpu.async_copy(
                            h_hbm.at[ebuf.at[cur + RB, 0]], rows.at[b], sem
                        )

        @pl.when(wid < XTRA)
        def _():
            pltpu.async_copy(h_hbm.at[xbuf.at[0, 0]], rows.at[0], sem).wait()
            pltpu.sync_copy(rows.at[0], acc.at[xbuf.at[0, 1]], add=True)

        plsc.subcore_barrier()
        pltpu.sync_copy(acc.at[pl.ds(s * RPA, RPA)], out_hbm.at[c, pl.ds(s * RPA, RPA)])

        @pl.when(s == NS - 1)
        def _():
            pltpu.sync_copy(acc.at[pl.ds(NS * RPA, N - NS * RPA)],
                            out_hbm.at[c, pl.ds(NS * RPA, N - NS * RPA)])

    return k(h, ev)


def _tc_dis(histp):
    def body(hp_ref, dis_ref):
        hp = hp_ref[...].reshape(NW, NBINS // 128, 128)
        deg = jnp.sum(hp, axis=0) + 1.0
        dis_ref[...] = lax.rsqrt(deg)

    return pl.pallas_call(
        body,
        out_shape=jax.ShapeDtypeStruct((NBINS // 128, 128), jnp.float32),
    )(histp)


def _dis_packed(dis2_ref):
    d = dis2_ref[...]
    lo = jnp.broadcast_to(d[:, 0:1], (NP, D_HID))
    hi = jnp.broadcast_to(d[:, 1:2], (NP, D_HID))
    return jnp.concatenate([lo, hi], axis=1)


def _tc_layer1(dis2, xp, W1p):
    def body(dis2_ref, xp_ref, w_ref, out_ref):
        out_ref[...] = _dis_packed(dis2_ref) * jnp.dot(
            xp_ref[...], w_ref[...], preferred_element_type=jnp.float32
        )

    return pl.pallas_call(
        body,
        out_shape=jax.ShapeDtypeStruct((NP, 128), jnp.float32),
    )(dis2, xp, W1p)


def _tc_layer2(aggp, h1p, dis2, b1p, W2p):
    def body(aggp_ref, h1p_ref, dis2_ref, b1p_ref, w_ref, out_ref):
        dis = _dis_packed(dis2_ref)
        agg = aggp_ref[0] + aggp_ref[1]
        h1 = jnp.maximum(dis * (agg + h1p_ref[...]) + b1p_ref[...], 0.0)
        out_ref[...] = dis * jnp.dot(
            h1, w_ref[...], preferred_element_type=jnp.float32
        )

    return pl.pallas_call(
        body,
        out_shape=jax.ShapeDtypeStruct((NP, 128), jnp.float32),
    )(aggp, h1p, dis2, b1p, W2p)


def _tc_final(aggp, h2p, dis2, b2p, batch2, linW, linb):
    def body(aggp_ref, h2p_ref, dis2_ref, b2p_ref, batch_ref, lw_ref, lb_ref,
             out_ref):
        dis = _dis_packed(dis2_ref)
        agg = aggp_ref[0] + aggp_ref[1]
        h2 = jnp.maximum(dis * (agg + h2p_ref[...]) + b2p_ref[...], 0.0)
        gid = lax.broadcasted_iota(jnp.int32, (NG, NP), 0)
        one_e = (batch_ref[0:1, :] == gid).astype(jnp.float32)
        one_o = (batch_ref[1:2, :] == gid).astype(jnp.float32)
        pooled = (
            jnp.dot(one_e, h2[:, :D_HID], preferred_element_type=jnp.float32)
            + jnp.dot(one_o, h2[:, D_HID:], preferred_element_type=jnp.float32)
        )
        counts = jnp.sum(one_e, axis=1, keepdims=True) + jnp.sum(
            one_o, axis=1, keepdims=True
        )
        mean = pooled / jnp.maximum(counts, 1.0)
        logits = jnp.dot(mean, lw_ref[...], preferred_element_type=jnp.float32)
        logits = logits + lb_ref[...]
        m = jnp.max(logits, axis=1, keepdims=True)
        lse = m + jnp.log(jnp.sum(jnp.exp(logits - m), axis=1, keepdims=True))
        out_ref[...] = logits - lse

    return pl.pallas_call(
        body,
        out_shape=jax.ShapeDtypeStruct((NG, 2), jnp.float32),
    )(aggp, h2p, dis2, b2p, batch2, linW, linb)


def _block_diag2(W):
    a, b = W.shape
    z = jnp.zeros((a, b), W.dtype)
    return jnp.concatenate(
        [jnp.concatenate([W, z], axis=1), jnp.concatenate([z, W], axis=1)],
        axis=0,
    )


def kernel(x, edge_index, batch, W1, b1, W2, b2, linW, linb):
    ev = jnp.transpose(
        edge_index.astype(jnp.int32).reshape(2, NCHUNK, CH), (1, 0, 2)
    )

    histp = _deg_hist(ev)
    dis80 = _tc_dis(histp.reshape(NW * NBINS // 128, 128))
    dis2 = dis80.reshape(NBINS)[:N].reshape(NP, 2)

    xp = x.reshape(NP, 2 * 128)
    W1p = _block_diag2(W1)
    W2p = _block_diag2(W2)
    b1p = jnp.concatenate([b1, b1]).reshape(1, 128)
    b2p = jnp.concatenate([b2, b2]).reshape(1, 128)
    batch2 = batch.astype(jnp.int32).reshape(NP, 2).T

    h1p = _tc_layer1(dis2, xp, W1p)
    agg1 = _edge_agg(h1p.reshape(N, D_HID), ev)
    agg1 = agg1.reshape(NC, NP, 128)
    h2p = _tc_layer2(agg1, h1p, dis2, b1p, W2p)
    agg2 = _edge_agg(h2p.reshape(N, D_HID), ev)
    agg2 = agg2.reshape(NC, NP, 128)
    out = _tc_final(agg2, h2p, dis2, b2p, batch2, linW, linb)
    return out


# speedup vs baseline: 1.0088x; 1.0068x over previous
import functools

import jax
import jax.numpy as jnp
from jax import lax
from jax.experimental import pallas as pl
from jax.experimental.pallas import tpu as pltpu
from jax.experimental.pallas import tpu_sc as plsc

N = 10000
NP = N // 2
E = 320000
D_HID = 64
NG = 16
NC, NS = 2, 16
NW = NC * NS
CH = 128
NCHUNK = E // CH
CPT = NCHUNK // NW
XTRA = NCHUNK - NW * CPT
RB = 8
RPA = 624
NBINS = 10240


def _sc_mesh():
    return plsc.VectorSubcoreMesh(core_axis_name="c", subcore_axis_name="s")


def _deg_hist(ev):
    @functools.partial(
        pl.kernel,
        out_type=jax.ShapeDtypeStruct((NW, NBINS), jnp.float32),
        mesh=_sc_mesh(),
        scratch_types=[
            pltpu.VMEM((CPT, 2, CH), jnp.int32),
            pltpu.VMEM((1, 2, CH), jnp.int32),
            pltpu.VMEM((NBINS,), jnp.float32),
        ],
        compiler_params=pltpu.CompilerParams(
            needs_layout_passes=False, use_tc_tiling_on_sc=False
        ),
    )
    def k(ev_hbm, out_hbm, ebuf, xbuf, hist):
        c = lax.axis_index("c")
        s = lax.axis_index("s")
        wid = c * NS + s
        pltpu.sync_copy(ev_hbm.at[pl.ds(wid * CPT, CPT)], ebuf)

        @pl.when(wid < XTRA)
        def _():
            pltpu.sync_copy(ev_hbm.at[pl.ds(NW * CPT + wid, 1)], xbuf)

        zeros = jnp.zeros((16,), jnp.float32)

        @pl.loop(0, NBINS // 16)
        def _(r):
            hist[pl.ds(r * 16, 16)] = zeros

        ones = jnp.ones((16,), jnp.float32)

        @pl.loop(0, CPT)
        def _(g):
            for j in range(CH // 16):
                idx = ebuf[g, 1, pl.ds(j * 16, 16)]
                plsc.addupdate_scatter(hist, [idx], ones)

        @pl.when(wid < XTRA)
        def _():
            for j in range(CH // 16):
                idx = xbuf[0, 1, pl.ds(j * 16, 16)]
                plsc.addupdate_scatter(hist, [idx], ones)

        pltpu.sync_copy(hist, out_hbm.at[wid])

    return k(ev)


def _edge_agg(h, ev):
    @functools.partial(
        pl.kernel,
        out_type=jax.ShapeDtypeStruct((NC, N, D_HID), jnp.float32),
        mesh=_sc_mesh(),
        scratch_types=[
            pltpu.VMEM((CPT + 1, 2, CH), jnp.int32),
            pltpu.VMEM((RB, CH, D_HID), jnp.float32),
            pltpu.VMEM((48, D_HID), jnp.float32),
            pltpu.VMEM_SHARED((N, D_HID), jnp.float32),
            pltpu.SemaphoreType.DMA,
        ],
        compiler_params=pltpu.CompilerParams(
            needs_layout_passes=False, use_tc_tiling_on_sc=False
        ),
    )
    def k(h_hbm, ev_hbm, out_hbm, ebuf, rows, zbuf, acc, sem):
        c = lax.axis_index("c")
        s = lax.axis_index("s")
        wid = c * NS + s
        ncpt = CPT + jnp.where(wid < XTRA, 1, 0)
        pltpu.sync_copy(ev_hbm.at[pl.ds(wid * CPT, CPT)], ebuf.at[pl.ds(0, CPT)])

        @pl.when(wid < XTRA)
        def _():
            pltpu.sync_copy(ev_hbm.at[pl.ds(NW * CPT + wid, 1)],
                            ebuf.at[pl.ds(CPT, 1)])

        zeros = jnp.zeros((16,), jnp.float32)

        @pl.loop(0, 48)
        def _(r):
            for j in range(D_HID // 16):
                zbuf[r, pl.ds(j * 16, 16)] = zeros

        for j in range(13):
            pltpu.async_copy(zbuf, acc.at[pl.ds(s * RPA + j * 48, 48)], sem)

        @pl.when(s == NS - 1)
        def _():
            pltpu.async_copy(zbuf.at[pl.ds(0, N - NS * RPA)],
                            acc.at[pl.ds(NS * RPA, N - NS * RPA)], sem)

        for j in range(13):
            pltpu.make_async_copy(
                zbuf, acc.at[pl.ds(s * RPA + j * 48, 48)], sem
            ).wait()

        @pl.when(s == NS - 1)
        def _():
            pltpu.make_async_copy(
                zbuf.at[pl.ds(0, N - NS * RPA)],
                acc.at[pl.ds(NS * RPA, N - NS * RPA)], sem
            ).wait()

        plsc.subcore_barrier()

        for b in range(RB):
            pltpu.async_copy(h_hbm.at[ebuf.at[b, 0]], rows.at[b], sem)

        @pl.loop(0, CPT + 1, step=RB)
        def _(g):
            for b in range(RB):
                cur = g + b

                @pl.when(cur < ncpt)
                def _():
                    pltpu.make_async_copy(
                        h_hbm.at[ebuf.at[cur, 0]], rows.at[b], sem
                    ).wait()
                    pltpu.sync_copy(rows.at[b], acc.at[ebuf.at[cur, 1]], add=True)

                    @pl.when(cur + RB < ncpt)
                    def _():
                        pltpu.async_copy(
                            h_hbm.at[ebuf.at[cur + RB, 0]], rows.at[b], sem
                        )

        plsc.subcore_barrier()
        pltpu.sync_copy(acc.at[pl.ds(s * RPA, RPA)], out_hbm.at[c, pl.ds(s * RPA, RPA)])

        @pl.when(s == NS - 1)
        def _():
            pltpu.sync_copy(acc.at[pl.ds(NS * RPA, N - NS * RPA)],
                            out_hbm.at[c, pl.ds(NS * RPA, N - NS * RPA)])

    return k(h, ev)


def _tc_dis(histp):
    def body(hp_ref, dis_ref):
        hp = hp_ref[...].reshape(NW, NBINS // 128, 128)
        deg = jnp.sum(hp, axis=0) + 1.0
        dis_ref[...] = lax.rsqrt(deg)

    return pl.pallas_call(
        body,
        out_shape=jax.ShapeDtypeStruct((NBINS // 128, 128), jnp.float32),
    )(histp)


def _dis_packed(dis2_ref):
    d = dis2_ref[...]
    lo = jnp.broadcast_to(d[:, 0:1], (NP, D_HID))
    hi = jnp.broadcast_to(d[:, 1:2], (NP, D_HID))
    return jnp.concatenate([lo, hi], axis=1)


def _tc_layer1(dis2, xp, W1p):
    def body(dis2_ref, xp_ref, w_ref, out_ref):
        out_ref[...] = _dis_packed(dis2_ref) * jnp.dot(
            xp_ref[...], w_ref[...], preferred_element_type=jnp.float32
        )

    return pl.pallas_call(
        body,
        out_shape=jax.ShapeDtypeStruct((NP, 128), jnp.float32),
    )(dis2, xp, W1p)


def _tc_layer2(aggp, h1p, dis2, b1p, W2p):
    def body(aggp_ref, h1p_ref, dis2_ref, b1p_ref, w_ref, out_ref):
        dis = _dis_packed(dis2_ref)
        agg = aggp_ref[0] + aggp_ref[1]
        h1 = jnp.maximum(dis * (agg + h1p_ref[...]) + b1p_ref[...], 0.0)
        out_ref[...] = dis * jnp.dot(
            h1, w_ref[...], preferred_element_type=jnp.float32
        )

    return pl.pallas_call(
        body,
        out_shape=jax.ShapeDtypeStruct((NP, 128), jnp.float32),
    )(aggp, h1p, dis2, b1p, W2p)


def _tc_final(aggp, h2p, dis2, b2p, batch2, linW, linb):
    def body(aggp_ref, h2p_ref, dis2_ref, b2p_ref, batch_ref, lw_ref, lb_ref,
             out_ref):
        dis = _dis_packed(dis2_ref)
        agg = aggp_ref[0] + aggp_ref[1]
        h2 = jnp.maximum(dis * (agg + h2p_ref[...]) + b2p_ref[...], 0.0)
        gid = lax.broadcasted_iota(jnp.int32, (NG, NP), 0)
        one_e = (batch_ref[0:1, :] == gid).astype(jnp.float32)
        one_o = (batch_ref[1:2, :] == gid).astype(jnp.float32)
        pooled = (
            jnp.dot(one_e, h2[:, :D_HID], preferred_element_type=jnp.float32)
            + jnp.dot(one_o, h2[:, D_HID:], preferred_element_type=jnp.float32)
        )
        counts = jnp.sum(one_e, axis=1, keepdims=True) + jnp.sum(
            one_o, axis=1, keepdims=True
        )
        mean = pooled / jnp.maximum(counts, 1.0)
        logits = jnp.dot(mean, lw_ref[...], preferred_element_type=jnp.float32)
        logits = logits + lb_ref[...]
        m = jnp.max(logits, axis=1, keepdims=True)
        lse = m + jnp.log(jnp.sum(jnp.exp(logits - m), axis=1, keepdims=True))
        out_ref[...] = logits - lse

    return pl.pallas_call(
        body,
        out_shape=jax.ShapeDtypeStruct((NG, 2), jnp.float32),
    )(aggp, h2p, dis2, b2p, batch2, linW, linb)


def _block_diag2(W):
    a, b = W.shape
    z = jnp.zeros((a, b), W.dtype)
    return jnp.concatenate(
        [jnp.concatenate([W, z], axis=1), jnp.concatenate([z, W], axis=1)],
        axis=0,
    )


def kernel(x, edge_index, batch, W1, b1, W2, b2, linW, linb):
    ev = jnp.transpose(
        edge_index.astype(jnp.int32).reshape(2, NCHUNK, CH), (1, 0, 2)
    )

    histp = _deg_hist(ev)
    dis80 = _tc_dis(histp.reshape(NW * NBINS // 128, 128))
    dis2 = dis80.reshape(NBINS)[:N].reshape(NP, 2)

    xp = x.reshape(NP, 2 * 128)
    W1p = _block_diag2(W1)
    W2p = _block_diag2(W2)
    b1p = jnp.concatenate([b1, b1]).reshape(1, 128)
    b2p = jnp.concatenate([b2, b2]).reshape(1, 128)
    batch2 = batch.astype(jnp.int32).reshape(NP, 2).T

    h1p = _tc_layer1(dis2, xp, W1p)
    agg1 = _edge_agg(h1p.reshape(N, D_HID), ev)
    agg1 = agg1.reshape(NC, NP, 128)
    h2p = _tc_layer2(agg1, h1p, dis2, b1p, W2p)
    agg2 = _edge_agg(h2p.reshape(N, D_HID), ev)
    agg2 = agg2.reshape(NC, NP, 128)
    out = _tc_final(agg2, h2p, dis2, b2p, batch2, linW, linb)
    return out


# speedup vs baseline: 1.0109x; 1.0021x over previous
import functools

import jax
import jax.numpy as jnp
from jax import lax
from jax.experimental import pallas as pl
from jax.experimental.pallas import tpu as pltpu
from jax.experimental.pallas import tpu_sc as plsc

N = 10000
NP = N // 2
E = 320000
D_HID = 64
NG = 16
NC, NS = 2, 16
NW = NC * NS
CH = 128
NCHUNK = E // CH
CPT = NCHUNK // NW
XTRA = NCHUNK - NW * CPT
RB = 8
RPA = 624
NBINS = 10240


def _sc_mesh():
    return plsc.VectorSubcoreMesh(core_axis_name="c", subcore_axis_name="s")


def _deg_hist(ev):
    @functools.partial(
        pl.kernel,
        out_type=jax.ShapeDtypeStruct((NW, NBINS), jnp.float32),
        mesh=_sc_mesh(),
        scratch_types=[
            pltpu.VMEM((CPT + 1, 1, CH), jnp.int32),
            pltpu.VMEM((NBINS,), jnp.float32),
        ],
        compiler_params=pltpu.CompilerParams(
            needs_layout_passes=False, use_tc_tiling_on_sc=False
        ),
    )
    def k(ev_hbm, out_hbm, dbuf, hist):
        c = lax.axis_index("c")
        s = lax.axis_index("s")
        wid = c * NS + s
        pltpu.sync_copy(ev_hbm.at[pl.ds(wid * CPT, CPT), pl.ds(1, 1)],
                        dbuf.at[pl.ds(0, CPT)])

        @pl.when(wid < XTRA)
        def _():
            pltpu.sync_copy(ev_hbm.at[pl.ds(NW * CPT + wid, 1), pl.ds(1, 1)],
                            dbuf.at[pl.ds(CPT, 1)])

        zeros = jnp.zeros((16,), jnp.float32)

        @pl.loop(0, NBINS // 16)
        def _(r):
            hist[pl.ds(r * 16, 16)] = zeros

        ones = jnp.ones((16,), jnp.float32)
        ncpt = CPT + jnp.where(wid < XTRA, 1, 0)

        @pl.loop(0, CPT + 1)
        def _(g):
            @pl.when(g < ncpt)
            def _():
                for j in range(CH // 16):
                    idx = dbuf[g, 0, pl.ds(j * 16, 16)]
                    plsc.addupdate_scatter(hist, [idx], ones)

        pltpu.sync_copy(hist, out_hbm.at[wid])

    return k(ev)


def _edge_agg(h, ev):
    @functools.partial(
        pl.kernel,
        out_type=jax.ShapeDtypeStruct((NC, N, D_HID), jnp.float32),
        mesh=_sc_mesh(),
        scratch_types=[
            pltpu.VMEM((CPT + 1, 2, CH), jnp.int32),
            pltpu.VMEM((RB, CH, D_HID), jnp.float32),
            pltpu.VMEM((48, D_HID), jnp.float32),
            pltpu.VMEM_SHARED((N, D_HID), jnp.float32),
            pltpu.SemaphoreType.DMA,
        ],
        compiler_params=pltpu.CompilerParams(
            needs_layout_passes=False, use_tc_tiling_on_sc=False
        ),
    )
    def k(h_hbm, ev_hbm, out_hbm, ebuf, rows, zbuf, acc, sem):
        c = lax.axis_index("c")
        s = lax.axis_index("s")
        wid = c * NS + s
        ncpt = CPT + jnp.where(wid < XTRA, 1, 0)
        pltpu.sync_copy(ev_hbm.at[pl.ds(wid * CPT, CPT)], ebuf.at[pl.ds(0, CPT)])

        @pl.when(wid < XTRA)
        def _():
            pltpu.sync_copy(ev_hbm.at[pl.ds(NW * CPT + wid, 1)],
                            ebuf.at[pl.ds(CPT, 1)])

        zeros = jnp.zeros((16,), jnp.float32)

        @pl.loop(0, 48)
        def _(r):
            for j in range(D_HID // 16):
                zbuf[r, pl.ds(j * 16, 16)] = zeros

        for j in range(13):
            pltpu.async_copy(zbuf, acc.at[pl.ds(s * RPA + j * 48, 48)], sem)

        @pl.when(s == NS - 1)
        def _():
            pltpu.async_copy(zbuf.at[pl.ds(0, N - NS * RPA)],
                            acc.at[pl.ds(NS * RPA, N - NS * RPA)], sem)

        for j in range(13):
            pltpu.make_async_copy(
                zbuf, acc.at[pl.ds(s * RPA + j * 48, 48)], sem
            ).wait()

        @pl.when(s == NS - 1)
        def _():
            pltpu.make_async_copy(
                zbuf.at[pl.ds(0, N - NS * RPA)],
                acc.at[pl.ds(NS * RPA, N - NS * RPA)], sem
            ).wait()

        plsc.subcore_barrier()

        for b in range(RB):
            pltpu.async_copy(h_hbm.at[ebuf.at[b, 0]], rows.at[b], sem)

        @pl.loop(0, CPT + 1, step=RB)
        def _(g):
            for b in range(RB):
                cur = g + b

                @pl.when(cur < ncpt)
                def _():
                    pltpu.make_async_copy(
                        h_hbm.at[ebuf.at[cur, 0]], rows.at[b], sem
                    ).wait()
                    pltpu.sync_copy(rows.at[b], acc.at[ebuf.at[cur, 1]], add=True)

                    @pl.when(cur + RB < ncpt)
                    def _():
                        pltpu.async_copy(
                            h_hbm.at[ebuf.at[cur + RB, 0]], rows.at[b], sem
                        )

        plsc.subcore_barrier()
        pltpu.sync_copy(acc.at[pl.ds(s * RPA, RPA)], out_hbm.at[c, pl.ds(s * RPA, RPA)])

        @pl.when(s == NS - 1)
        def _():
            pltpu.sync_copy(acc.at[pl.ds(NS * RPA, N - NS * RPA)],
                            out_hbm.at[c, pl.ds(NS * RPA, N - NS * RPA)])

    return k(h, ev)


def _tc_dis(histp):
    def body(hp_ref, dis_ref):
        hp = hp_ref[...].reshape(NW, NBINS // 128, 128)
        deg = jnp.sum(hp, axis=0) + 1.0
        dis_ref[...] = lax.rsqrt(deg)

    return pl.pallas_call(
        body,
        out_shape=jax.ShapeDtypeStruct((NBINS // 128, 128), jnp.float32),
    )(histp)


def _dis_packed(dis2_ref):
    d = dis2_ref[...]
    lo = jnp.broadcast_to(d[:, 0:1], (NP, D_HID))
    hi = jnp.broadcast_to(d[:, 1:2], (NP, D_HID))
    return jnp.concatenate([lo, hi], axis=1)


def _tc_layer1(dis2, xp, W1p):
    def body(dis2_ref, xp_ref, w_ref, out_ref):
        out_ref[...] = _dis_packed(dis2_ref) * jnp.dot(
            xp_ref[...], w_ref[...], preferred_element_type=jnp.float32
        )

    return pl.pallas_call(
        body,
        out_shape=jax.ShapeDtypeStruct((NP, 128), jnp.float32),
    )(dis2, xp, W1p)


def _tc_layer2(aggp, h1p, dis2, b1p, W2p):
    def body(aggp_ref, h1p_ref, dis2_ref, b1p_ref, w_ref, out_ref):
        dis = _dis_packed(dis2_ref)
        agg = aggp_ref[0] + aggp_ref[1]
        h1 = jnp.maximum(dis * (agg + h1p_ref[...]) + b1p_ref[...], 0.0)
        out_ref[...] = dis * jnp.dot(
            h1, w_ref[...], preferred_element_type=jnp.float32
        )

    return pl.pallas_call(
        body,
        out_shape=jax.ShapeDtypeStruct((NP, 128), jnp.float32),
    )(aggp, h1p, dis2, b1p, W2p)


def _tc_final(aggp, h2p, dis2, b2p, batch2, linW, linb):
    def body(aggp_ref, h2p_ref, dis2_ref, b2p_ref, batch_ref, lw_ref, lb_ref,
             out_ref):
        dis = _dis_packed(dis2_ref)
        agg = aggp_ref[0] + aggp_ref[1]
        h2 = jnp.maximum(dis * (agg + h2p_ref[...]) + b2p_ref[...], 0.0)
        gid = lax.broadcasted_iota(jnp.int32, (NG, NP), 0)
        one_e = (batch_ref[0:1, :] == gid).astype(jnp.float32)
        one_o = (batch_ref[1:2, :] == gid).astype(jnp.float32)
        pooled = (
            jnp.dot(one_e, h2[:, :D_HID], preferred_element_type=jnp.float32)
            + jnp.dot(one_o, h2[:, D_HID:], preferred_element_type=jnp.float32)
        )
        counts = jnp.sum(one_e, axis=1, keepdims=True) + jnp.sum(
            one_o, axis=1, keepdims=True
        )
        mean = pooled / jnp.maximum(counts, 1.0)
        logits = jnp.dot(mean, lw_ref[...], preferred_element_type=jnp.float32)
        logits = logits + lb_ref[...]
        m = jnp.max(logits, axis=1, keepdims=True)
        lse = m + jnp.log(jnp.sum(jnp.exp(logits - m), axis=1, keepdims=True))
        out_ref[...] = logits - lse

    return pl.pallas_call(
        body,
        out_shape=jax.ShapeDtypeStruct((NG, 2), jnp.float32),
    )(aggp, h2p, dis2, b2p, batch2, linW, linb)


def _block_diag2(W):
    a, b = W.shape
    z = jnp.zeros((a, b), W.dtype)
    return jnp.concatenate(
        [jnp.concatenate([W, z], axis=1), jnp.concatenate([z, W], axis=1)],
        axis=0,
    )


def kernel(x, edge_index, batch, W1, b1, W2, b2, linW, linb):
    ev = jnp.transpose(
        edge_index.astype(jnp.int32).reshape(2, NCHUNK, CH), (1, 0, 2)
    )

    histp = _deg_hist(ev)
    dis80 = _tc_dis(histp.reshape(NW * NBINS // 128, 128))
    dis2 = dis80.reshape(NBINS)[:N].reshape(NP, 2)

    xp = x.reshape(NP, 2 * 128)
    W1p = _block_diag2(W1)
    W2p = _block_diag2(W2)
    b1p = jnp.concatenate([b1, b1]).reshape(1, 128)
    b2p = jnp.concatenate([b2, b2]).reshape(1, 128)
    batch2 = batch.astype(jnp.int32).reshape(NP, 2).T

    h1p = _tc_layer1(dis2, xp, W1p)
    agg1 = _edge_agg(h1p.reshape(N, D_HID), ev)
    agg1 = agg1.reshape(NC, NP, 128)
    h2p = _tc_layer2(agg1, h1p, dis2, b1p, W2p)
    agg2 = _edge_agg(h2p.reshape(N, D_HID), ev)
    agg2 = agg2.reshape(NC, NP, 128)
    out = _tc_final(agg2, h2p, dis2, b2p, batch2, linW, linb)
    return out
